# baseline (speedup 1.0000x reference)
import functools

import jax
import jax.numpy as jnp
from jax import lax
from jax.experimental import pallas as pl
from jax.experimental.pallas import tpu as pltpu
from jax.experimental.pallas import tpu_sc as plsc

NUM_HEADS = 8
INPUT_BITS = 2048
N_STATE = 2048
N_OUT = 256
NB_SKIP = 7
NB_USED = 13
NB_OUT = 8
HASH = 8192
K_BITS = 8
NUM_WINDOWS = 32

NS = 16
L = 16
NPT = N_STATE // NS
G = NPT // L

W0 = 0
W1 = INPUT_BITS
ST = 2 * INPUT_BITS


def _body(ib, connp, sm, hc, hm, out, buf, conn_v, addr_a, addr_b, bits_v,
          hidx_v, hcv, vals_a, vals_b, outi_v, outv_v, shared, sem_w, sem_g,
          sem_h):
    c = lax.axis_index("c")
    s = lax.axis_index("s")
    iota = lax.iota(jnp.int32, L)

    @pl.when(c == 0)
    def _core0():
        _scan_and_heads(ib, connp, sm, hc, hm, out, buf, conn_v, addr_a,
                        addr_b, bits_v, hidx_v, hcv, vals_a, vals_b, outi_v,
                        outv_v, shared, sem_w, sem_g, sem_h, s, iota)


def _scan_and_heads(ib, connp, sm, hc, hm, out, buf, conn_v, addr_a, addr_b,
                    bits_v, hidx_v, hcv, vals_a, vals_b, outi_v, outv_v,
                    shared, sem_w, sem_g, sem_h, s, iota):
    pltpu.sync_copy(connp.at[s], conn_v)

    def _zero(i, carry):
        buf[pl.ds(ST + i * L, L)] = jnp.zeros((L,), jnp.int32)
        return carry

    lax.fori_loop(0, N_STATE // L, _zero, 0)

    def _window_dma(t, slot):
        return pltpu.async_copy(
            ib.at[pl.ds(t * INPUT_BITS, INPUT_BITS)],
            buf.at[pl.ds(slot, INPUT_BITS)], sem_w)

    def _window_wait():
        pltpu.make_async_copy(ib.at[pl.ds(0, INPUT_BITS)],
                              buf.at[pl.ds(W0, INPUT_BITS)], sem_w).wait()

    pltpu.sync_copy(ib.at[pl.ds(NUM_WINDOWS * INPUT_BITS - L, L)],
                    bits_v.at[pl.ds(0, L)])
    kb = bits_v[pl.ds(0, L)]
    w = jnp.where(iota >= L - K_BITS,
                  jnp.full((L,), 1, jnp.int32) << (L - 1 - iota),
                  jnp.zeros((L,), jnp.int32))
    hsel = jnp.bitwise_and(jnp.sum(kb * w), NUM_HEADS - 1)
    for j in range(NB_OUT):
        hidx_v[pl.ds(j * L, L)] = (hsel * 2048 + (s >> 3) * 1024 + j * 128
                                   + (s & 7) * L + iota)
    hc_handle = pltpu.async_copy(hc.at[hidx_v], hcv, sem_h)

    _window_dma(0, W0)

    def _half_step(p):
        def _addr(g, dst):
            acc = jnp.zeros((L,), jnp.int32)
            for j in range(NB_USED):
                idx = conn_v[p * NB_USED + j, pl.ds(g * L, L)]
                bit = plsc.load_gather(buf, [idx])
                acc = acc + acc + bit
            r = s * NPT + g * L + iota
            dst[pl.ds((g % (G // 2)) * L, L)] = (
                ((r >> 3) << 16) + ((acc >> 7) << 10)
                + ((r & 7) << 7) + (acc & 127))

        for g in range(G // 2):
            _addr(g, addr_a)
        ha = pltpu.async_copy(sm.at[addr_a], vals_a, sem_g)
        for g in range(G // 2, G):
            _addr(g, addr_b)
        hb = pltpu.async_copy(sm.at[addr_b], vals_b, sem_g)
        ha.wait()
        for g in range(G // 2):
            v = vals_a[pl.ds(g * L, L)]
            bits_v[pl.ds(g * L, L)] = (v > 0.5).astype(jnp.int32)
        hb.wait()
        for g in range(G // 2):
            v = vals_b[pl.ds(g * L, L)]
            bits_v[pl.ds((g + G // 2) * L, L)] = (v > 0.5).astype(jnp.int32)
        pltpu.sync_copy(bits_v,
                        shared.at[pl.ds(p * N_STATE + s * NPT, NPT)])
        plsc.subcore_barrier()
        pltpu.sync_copy(shared.at[pl.ds(p * N_STATE, N_STATE)],
                        buf.at[pl.ds(ST, N_STATE)])

    def _pair(k, carry):
        t0 = 2 * k
        _window_wait()
        _window_dma(t0 + 1, W1)
        _half_step(0)
        _window_wait()
        _window_dma(jnp.minimum(t0 + 2, NUM_WINDOWS - 1), W0)
        _half_step(1)
        return carry

    lax.fori_loop(0, NUM_WINDOWS // 2, _pair, 0)
    _window_wait()
    hc_handle.wait()

    acc = jnp.zeros((L,), jnp.int32)
    for j in range(NB_OUT):
        sidx = hcv[pl.ds(j * L, L)] + ST
        bit = plsc.load_gather(buf, [sidx])
        acc = acc + acc + bit
    outi_v[...] = (hsel * 65536 + (s * 2 + (iota >> 3)) * 2048
                   + ((acc >> 7) << 10) + ((iota & 7) << 7) + (acc & 127))
    pltpu.async_copy(hm.at[outi_v], outv_v, sem_g).wait()
    pltpu.sync_copy(outv_v, out.at[pl.ds(s * L, L)])


def kernel(input_bits, state_mem, head_mem, state_conn, head_conn):
    conn = state_conn[:, NB_SKIP:].reshape(NS, NPT, NB_USED)
    conn = conn.transpose(0, 2, 1).astype(jnp.int32)
    conn_e = jnp.where(conn < INPUT_BITS, conn, conn + INPUT_BITS)
    conn_o = conn + INPUT_BITS
    conn_p = jnp.concatenate([conn_e, conn_o], axis=1)

    sm_flat = (state_mem.reshape(N_STATE // 8, 8, HASH // 128, 128)
               .transpose(0, 2, 1, 3).reshape(-1))
    hc_flat = (head_conn.reshape(NUM_HEADS, 2, 128, NB_OUT)
               .transpose(0, 1, 3, 2).reshape(-1))
    hm_flat = (head_mem.reshape(NUM_HEADS, N_OUT // 8, 8, 2, 128)
               .transpose(0, 1, 3, 2, 4).reshape(-1))

    mesh = plsc.VectorSubcoreMesh(core_axis_name="c", subcore_axis_name="s")
    run = functools.partial(
        pl.kernel,
        out_type=jax.ShapeDtypeStruct((N_OUT,), jnp.float32),
        mesh=mesh,
        compiler_params=pltpu.CompilerParams(needs_layout_passes=False,
                                             skip_device_barrier=True),
        scratch_types=[
            pltpu.VMEM((2 * INPUT_BITS + N_STATE,), jnp.int32),
            pltpu.VMEM((2 * NB_USED, NPT), jnp.int32),
            pltpu.VMEM((NPT // 2,), jnp.int32),
            pltpu.VMEM((NPT // 2,), jnp.int32),
            pltpu.VMEM((NPT,), jnp.int32),
            pltpu.VMEM((NB_OUT * L,), jnp.int32),
            pltpu.VMEM((NB_OUT * L,), jnp.int32),
            pltpu.VMEM((NPT // 2,), jnp.float32),
            pltpu.VMEM((NPT // 2,), jnp.float32),
            pltpu.VMEM((L,), jnp.int32),
            pltpu.VMEM((L,), jnp.float32),
            pltpu.VMEM_SHARED((2 * N_STATE,), jnp.int32),
            pltpu.SemaphoreType.DMA,
            pltpu.SemaphoreType.DMA,
            pltpu.SemaphoreType.DMA,
        ],
    )(_body)
    return run(input_bits, conn_p, sm_flat, hc_flat, hm_flat)


# speedup vs baseline: 1.0054x; 1.0054x over previous
import functools

import jax
import jax.numpy as jnp
from jax import lax
from jax.experimental import pallas as pl
from jax.experimental.pallas import tpu as pltpu
from jax.experimental.pallas import tpu_sc as plsc

NUM_HEADS = 8
INPUT_BITS = 2048
N_STATE = 2048
N_OUT = 256
NB_SKIP = 7
NB_USED = 13
NB_OUT = 8
HASH = 8192
K_BITS = 8
NUM_WINDOWS = 32

NS = 16
L = 16
NPT = N_STATE // NS
G = NPT // L

W0 = 0
W1 = INPUT_BITS
ST = 2 * INPUT_BITS


def _body(ib, connp, sm, hc, hm, out, buf, conn_v, addr_a, addr_b, bits_v,
          hidx_v, hcv, vals_a, vals_b, outi_v, outv_v, shared, sem_w, sem_g,
          sem_h):
    c = lax.axis_index("c")
    s = lax.axis_index("s")
    iota = lax.iota(jnp.int32, L)

    @pl.when(c == 0)
    def _core0():
        _scan_and_heads(ib, connp, sm, hc, hm, out, buf, conn_v, addr_a,
                        addr_b, bits_v, hidx_v, hcv, vals_a, vals_b, outi_v,
                        outv_v, shared, sem_w, sem_g, sem_h, s, iota)


def _scan_and_heads(ib, connp, sm, hc, hm, out, buf, conn_v, addr_a, addr_b,
                    bits_v, hidx_v, hcv, vals_a, vals_b, outi_v, outv_v,
                    shared, sem_w, sem_g, sem_h, s, iota):
    pltpu.sync_copy(connp.at[s], conn_v)

    def _zero(i, carry):
        buf[pl.ds(ST + i * L, L)] = jnp.zeros((L,), jnp.int32)
        return carry

    lax.fori_loop(0, N_STATE // L, _zero, 0)

    def _window_dma(t, slot):
        return pltpu.async_copy(
            ib.at[pl.ds(t * INPUT_BITS, INPUT_BITS)],
            buf.at[pl.ds(slot, INPUT_BITS)], sem_w)

    def _window_wait():
        pltpu.make_async_copy(ib.at[pl.ds(0, INPUT_BITS)],
                              buf.at[pl.ds(W0, INPUT_BITS)], sem_w).wait()

    pltpu.sync_copy(ib.at[pl.ds(NUM_WINDOWS * INPUT_BITS - L, L)],
                    bits_v.at[pl.ds(0, L)])
    kb = bits_v[pl.ds(0, L)]
    w = jnp.where(iota >= L - K_BITS,
                  jnp.full((L,), 1, jnp.int32) << (L - 1 - iota),
                  jnp.zeros((L,), jnp.int32))
    hsel = jnp.bitwise_and(jnp.sum(kb * w), NUM_HEADS - 1)
    for j in range(NB_OUT):
        hidx_v[pl.ds(j * L, L)] = (hsel * 2048 + (s >> 3) * 1024 + j * 128
                                   + (s & 7) * L + iota)
    hc_handle = pltpu.async_copy(hc.at[hidx_v], hcv, sem_h)

    _window_dma(0, W0)

    def _half_step(p):
        def _addr(g, dst):
            acc = jnp.zeros((L,), jnp.int32)
            for j in range(NB_USED):
                idx = conn_v[p * NB_USED + j, pl.ds(g * L, L)]
                bit = plsc.load_gather(buf, [idx])
                acc = acc + acc + bit
            r = s * NPT + g * L + iota
            dst[pl.ds((g % (G // 2)) * L, L)] = (
                ((r >> 3) << 16) + ((acc >> 7) << 10)
                + ((r & 7) << 7) + (acc & 127))

        for g in range(G // 2):
            _addr(g, addr_a)
        ha = pltpu.async_copy(sm.at[addr_a], vals_a, sem_g)
        for g in range(G // 2, G):
            _addr(g, addr_b)
        hb = pltpu.async_copy(sm.at[addr_b], vals_b, sem_g)
        ha.wait()
        for g in range(G // 2):
            v = vals_a[pl.ds(g * L, L)]
            bits_v[pl.ds(g * L, L)] = (v > 0.5).astype(jnp.int32)
        hb.wait()
        for g in range(G // 2):
            v = vals_b[pl.ds(g * L, L)]
            bits_v[pl.ds((g + G // 2) * L, L)] = (v > 0.5).astype(jnp.int32)
        pltpu.sync_copy(bits_v,
                        shared.at[pl.ds(p * N_STATE + s * NPT, NPT)])
        plsc.subcore_barrier()
        pltpu.sync_copy(shared.at[pl.ds(p * N_STATE, N_STATE)],
                        buf.at[pl.ds(ST, N_STATE)])

    def _pair(k, carry):
        t0 = 2 * k
        _window_wait()
        _window_dma(t0 + 1, W1)
        _half_step(0)
        _window_wait()
        _window_dma(jnp.minimum(t0 + 2, NUM_WINDOWS - 1), W0)
        _half_step(1)
        return carry

    lax.fori_loop(0, NUM_WINDOWS // 2, _pair, 0)
    _window_wait()
    hc_handle.wait()

    acc = jnp.zeros((L,), jnp.int32)
    for j in range(NB_OUT):
        sidx = hcv[pl.ds(j * L, L)] + ST
        bit = plsc.load_gather(buf, [sidx])
        acc = acc + acc + bit
    outi_v[...] = (hsel * 65536 + (s * 2 + (iota >> 3)) * 2048
                   + ((acc >> 7) << 10) + ((iota & 7) << 7) + (acc & 127))
    pltpu.async_copy(hm.at[outi_v], outv_v, sem_g).wait()
    pltpu.sync_copy(outv_v, out.at[pl.ds(s * L, L)])


def kernel(input_bits, state_mem, head_mem, state_conn, head_conn):
    conn = state_conn[:, NB_SKIP:].reshape(NS, NPT, NB_USED)
    conn = conn.transpose(0, 2, 1).astype(jnp.int32)
    conn_e = jnp.where(conn < INPUT_BITS, conn, conn + INPUT_BITS)
    conn_o = conn + INPUT_BITS
    conn_p = jnp.concatenate([conn_e, conn_o], axis=1)

    sm_flat = (state_mem.reshape(N_STATE // 8, 8, HASH // 128, 128)
               .transpose(0, 2, 1, 3).reshape(-1))
    hc_flat = (head_conn.reshape(NUM_HEADS, 2, 128, NB_OUT)
               .transpose(0, 1, 3, 2).reshape(-1))
    hm_flat = (head_mem.reshape(NUM_HEADS, N_OUT // 8, 8, 2, 128)
               .transpose(0, 1, 3, 2, 4).reshape(-1))

    mesh = plsc.VectorSubcoreMesh(core_axis_name="c", subcore_axis_name="s")
    run = functools.partial(
        pl.kernel,
        out_type=jax.ShapeDtypeStruct((N_OUT,), jnp.float32),
        mesh=mesh,
        compiler_params=pltpu.CompilerParams(needs_layout_passes=False),
        scratch_types=[
            pltpu.VMEM((2 * INPUT_BITS + N_STATE,), jnp.int32),
            pltpu.VMEM((2 * NB_USED, NPT), jnp.int32),
            pltpu.VMEM((NPT // 2,), jnp.int32),
            pltpu.VMEM((NPT // 2,), jnp.int32),
            pltpu.VMEM((NPT,), jnp.int32),
            pltpu.VMEM((NB_OUT * L,), jnp.int32),
            pltpu.VMEM((NB_OUT * L,), jnp.int32),
            pltpu.VMEM((NPT // 2,), jnp.float32),
            pltpu.VMEM((NPT // 2,), jnp.float32),
            pltpu.VMEM((L,), jnp.int32),
            pltpu.VMEM((L,), jnp.float32),
            pltpu.VMEM_SHARED((2 * N_STATE,), jnp.int32),
            pltpu.SemaphoreType.DMA,
            pltpu.SemaphoreType.DMA,
            pltpu.SemaphoreType.DMA,
        ],
    )(_body)
    return run(input_bits, conn_p, sm_flat, hc_flat, hm_flat)
